# baseline (speedup 1.0000x reference)
import functools

import jax
import jax.numpy as jnp
from jax.experimental import pallas as pl
from jax.experimental.pallas import tpu as pltpu

_CHUNK = 400
_NMID = 24
_TAIL = (240, 96, 64)


def _l_copy(L_ref, L_buf, l_sems, start, rows, slot):
    return pltpu.make_async_copy(
        L_ref.at[pl.ds(start, rows), :],
        L_buf.at[slot, pl.ds(0, rows), :],
        l_sems.at[slot],
    )


def _out_copy(out_ref, out_buf, o_sems, start, rows, slot):
    return pltpu.make_async_copy(
        out_buf.at[slot, pl.ds(0, rows), :],
        out_ref.at[pl.ds(start, rows), :],
        o_sems.at[slot],
    )


def _fused_body(L_ref, F_ref, Wc_ref, bc_ref, out_ref,
                L_buf, l_sems, out_buf, o_sems, *, chunk, nmid, tail):
    n_tail = len(tail)
    tail_starts = []
    s = nmid * chunk
    for t in tail:
        tail_starts.append(s)
        s += t

    def compute(l_chunk, start, rows):
        x = jnp.dot(l_chunk, F_ref[...], preferred_element_type=jnp.float32)
        f_row = F_ref[pl.ds(start, rows), :]
        lhs = jnp.concatenate([f_row + x, x * f_row], axis=1)
        return (
            jnp.dot(lhs, Wc_ref[...], preferred_element_type=jnp.float32)
            + bc_ref[...]
        )

    for e in range(2):
        _l_copy(L_ref, L_buf, l_sems, e * chunk, chunk, e).start()

    def step(j, carry):
        slot = jax.lax.rem(j, 2)
        _l_copy(L_ref, L_buf, l_sems, j * chunk, chunk, slot).wait()
        res = compute(L_buf[slot, 0:chunk, :], j * chunk, chunk)

        @pl.when(j >= 2)
        def _wait_out():
            _out_copy(out_ref, out_buf, o_sems, (j - 2) * chunk, chunk, slot).wait()

        out_buf[slot, 0:chunk, :] = res
        _out_copy(out_ref, out_buf, o_sems, j * chunk, chunk, slot).start()

        @pl.when(j + 2 < nmid)
        def _next_mid():
            _l_copy(L_ref, L_buf, l_sems, (j + 2) * chunk, chunk, slot).start()

        for t in range(min(2, n_tail)):
            @pl.when(j + 2 == nmid + t)
            def _next_tail(t=t):
                _l_copy(L_ref, L_buf, l_sems, tail_starts[t], tail[t], slot).start()

        return carry

    jax.lax.fori_loop(0, nmid, step, 0)

    for t in range(n_tail):
        e = nmid + t
        slot = e % 2
        start, rows = tail_starts[t], tail[t]
        _l_copy(L_ref, L_buf, l_sems, start, rows, slot).wait()
        res = compute(L_buf[slot, 0:rows, :], start, rows)
        pe = e - 2
        p_start = pe * chunk if pe < nmid else tail_starts[pe - nmid]
        p_rows = chunk if pe < nmid else tail[pe - nmid]
        _out_copy(out_ref, out_buf, o_sems, p_start, p_rows, slot).wait()
        out_buf[slot, 0:rows, :] = res
        _out_copy(out_ref, out_buf, o_sems, start, rows, slot).start()
        nt = t + 2
        if nt < n_tail:
            _l_copy(L_ref, L_buf, l_sems, tail_starts[nt], tail[nt],
                    (nmid + nt) % 2).start()

    for t in range(max(0, n_tail - 2), n_tail):
        e = nmid + t
        _out_copy(out_ref, out_buf, o_sems, tail_starts[t], tail[t], e % 2).wait()


def kernel(lap_matrix, eye_matrix, features, W1, b1, W2, b2):
    n, d = features.shape
    chunk, nmid, tail = _CHUNK, _NMID, _TAIL
    assert nmid * chunk + sum(tail) == n

    Wc = jnp.concatenate([W1.T, W2.T], axis=0)
    bc = (b1 + b2).reshape(1, d)

    body = functools.partial(_fused_body, chunk=chunk, nmid=nmid, tail=tail)
    return pl.pallas_call(
        body,
        in_specs=[
            pl.BlockSpec(memory_space=pltpu.MemorySpace.HBM),
            pl.BlockSpec((n, d), lambda: (0, 0)),
            pl.BlockSpec((2 * d, d), lambda: (0, 0)),
            pl.BlockSpec((1, d), lambda: (0, 0)),
        ],
        out_specs=pl.BlockSpec(memory_space=pltpu.MemorySpace.HBM),
        out_shape=jax.ShapeDtypeStruct((n, d), jnp.float32),
        scratch_shapes=[
            pltpu.VMEM((2, chunk, n), jnp.float32),
            pltpu.SemaphoreType.DMA((2,)),
            pltpu.VMEM((2, chunk, d), jnp.float32),
            pltpu.SemaphoreType.DMA((2,)),
        ],
    )(lap_matrix, features, Wc, bc)
